# baseline (speedup 1.0000x reference)
import jax
import jax.numpy as jnp
from jax import lax
from jax.experimental import pallas as pl
from jax.experimental.pallas import tpu as pltpu
from jax.experimental.pallas import tpu_sc as plsc

NC, NS, L = 2, 16, 16
NW = NC * NS

VOCAB = 1_000_000
D = 64
IW = 128
R = 4


def _gather_body(table_hbm, idx_hbm, out_hbm, idx_v, rows_v, sem):
    n_rows = idx_hbm.shape[0]
    rows_per_w = n_rows // NW
    n_chunks = rows_per_w // R
    wid = lax.axis_index("s") * NC + lax.axis_index("c")
    base = wid * rows_per_w

    def chunk(g, _):
        row0 = base + g * R
        pltpu.sync_copy(idx_hbm.at[pl.ds(row0, R)], idx_v)
        copies = [
            pltpu.async_copy(table_hbm.at[idx_v.at[j]], rows_v.at[j], sem)
            for j in range(R)
        ]
        for c in copies:
            c.wait()
        pltpu.sync_copy(rows_v, out_hbm.at[pl.ds(row0, R)])
        return ()

    lax.fori_loop(0, n_chunks, chunk, (), unroll=False)


def kernel(token_ids, table):
    B, S = token_ids.shape
    n_idx = B * S
    assert n_idx % (IW * NW * R) == 0
    n_rows = n_idx // IW
    idx2d = token_ids.reshape(n_rows, IW).astype(jnp.int32)

    mesh = plsc.VectorSubcoreMesh(core_axis_name="c", subcore_axis_name="s")
    gather = pl.kernel(
        _gather_body,
        out_type=jax.ShapeDtypeStruct((n_rows, IW, D), jnp.float32),
        mesh=mesh,
        scratch_types=[
            pltpu.VMEM((R, IW), jnp.int32),
            pltpu.VMEM((R, IW, D), jnp.float32),
            pltpu.SemaphoreType.DMA,
        ],
        compiler_params=pltpu.CompilerParams(use_tc_tiling_on_sc=False),
    )
    out = gather(table, idx2d)
    return out.reshape(B, S, D)


# speedup vs baseline: 1.0293x; 1.0293x over previous
import jax
import jax.numpy as jnp
from jax import lax
from jax.experimental import pallas as pl
from jax.experimental.pallas import tpu as pltpu
from jax.experimental.pallas import tpu_sc as plsc

NC, NS, L = 2, 16, 16
NW = NC * NS

D = 64
IW = 128
R = 4
NBUF = 2


def _gather_body(table_hbm, idx_hbm, out_hbm, idx_v, rows_v, sem0, sem1):
    n_rows = idx_hbm.shape[0]
    rows_per_w = n_rows // NW
    n_chunks = rows_per_w // R
    wid = lax.axis_index("s") * NC + lax.axis_index("c")
    base = wid * rows_per_w
    sems = (sem0, sem1)

    def stage(g, b):
        row0 = base + g * R
        pltpu.sync_copy(idx_hbm.at[pl.ds(row0, R)], idx_v.at[b])
        for j in range(R):
            pltpu.async_copy(table_hbm.at[idx_v.at[b, j]], rows_v.at[b, j],
                             sems[b])

    stage(0, 0)

    def pair(t, _):
        for b in range(NBUF):
            g = NBUF * t + b
            nb = 1 - b

            @pl.when(g + 1 < n_chunks)
            def _():
                stage(g + 1, nb)

            pltpu.make_async_copy(out_hbm.at[pl.ds(0, R)], rows_v.at[b],
                                  sems[b]).wait()
            pltpu.sync_copy(rows_v.at[b], out_hbm.at[pl.ds(base + g * R, R)])
        return ()

    lax.fori_loop(0, n_chunks // NBUF, pair, (), unroll=False)


def kernel(token_ids, table):
    B, S = token_ids.shape
    n_idx = B * S
    assert n_idx % (IW * NW * R * NBUF) == 0
    n_rows = n_idx // IW
    idx2d = token_ids.reshape(n_rows, IW).astype(jnp.int32)

    mesh = plsc.VectorSubcoreMesh(core_axis_name="c", subcore_axis_name="s")
    gather = pl.kernel(
        _gather_body,
        out_type=jax.ShapeDtypeStruct((n_rows, IW, D), jnp.float32),
        mesh=mesh,
        scratch_types=[
            pltpu.VMEM((NBUF, R, IW), jnp.int32),
            pltpu.VMEM((NBUF, R, IW, D), jnp.float32),
            pltpu.SemaphoreType.DMA,
            pltpu.SemaphoreType.DMA,
        ],
        compiler_params=pltpu.CompilerParams(use_tc_tiling_on_sc=False),
    )
    out = gather(table, idx2d)
    return out.reshape(B, S, D)


# speedup vs baseline: 1.2647x; 1.2288x over previous
import jax
import jax.numpy as jnp
from jax import lax
from jax.experimental import pallas as pl
from jax.experimental.pallas import tpu as pltpu
from jax.experimental.pallas import tpu_sc as plsc

NC, NS, L = 2, 16, 16
NW = NC * NS

D = 64
DP = 128
IW = 128
R = 2
NBUF = 2


def _gather_body(table_hbm, idx_hbm, out_hbm, idx_v, rows_v, sem0, sem1):
    n_rows = idx_hbm.shape[0]
    rows_per_w = n_rows // NW
    n_chunks = rows_per_w // R
    wid = lax.axis_index("s") * NC + lax.axis_index("c")
    base = wid * rows_per_w
    sems = (sem0, sem1)

    def stage(g, b):
        row0 = base + g * R
        pltpu.sync_copy(idx_hbm.at[pl.ds(row0, R)], idx_v.at[b])
        for j in range(R):
            pltpu.async_copy(table_hbm.at[idx_v.at[b, j]], rows_v.at[b, j],
                             sems[b])

    stage(0, 0)

    def pair(t, _):
        for b in range(NBUF):
            g = NBUF * t + b
            nb = 1 - b

            @pl.when(g + 1 < n_chunks)
            def _():
                stage(g + 1, nb)

            pltpu.make_async_copy(table_hbm.at[idx_v.at[b]], rows_v.at[b],
                                  sems[b]).wait()
            pltpu.sync_copy(rows_v.at[b], out_hbm.at[pl.ds(base + g * R, R)])
        return ()

    lax.fori_loop(0, n_chunks // NBUF, pair, (), unroll=False)


def kernel(token_ids, table):
    B, S = token_ids.shape
    n_idx = B * S
    assert n_idx % (IW * NW * R * NBUF) == 0
    n_rows = n_idx // IW
    idx2d = token_ids.reshape(n_rows, IW).astype(jnp.int32)
    table_p = jnp.pad(table, ((0, 0), (0, DP - D)))

    mesh = plsc.VectorSubcoreMesh(core_axis_name="c", subcore_axis_name="s")
    gather = pl.kernel(
        _gather_body,
        out_type=jax.ShapeDtypeStruct((n_rows, IW, DP), jnp.float32),
        mesh=mesh,
        scratch_types=[
            pltpu.VMEM((NBUF, R, IW), jnp.int32),
            pltpu.VMEM((NBUF, R, IW, DP), jnp.float32),
            pltpu.SemaphoreType.DMA,
            pltpu.SemaphoreType.DMA,
        ],
        compiler_params=pltpu.CompilerParams(use_tc_tiling_on_sc=True),
    )
    out = gather(table_p, idx2d)
    return out[:, :, :D].reshape(B, S, D)


# speedup vs baseline: 1.3648x; 1.0791x over previous
import jax
import jax.numpy as jnp
from jax import lax
from jax.experimental import pallas as pl
from jax.experimental.pallas import tpu as pltpu
from jax.experimental.pallas import tpu_sc as plsc

NC, NS, L = 2, 16, 16
NW = NC * NS

D = 64
DP = 128
IW = 128
R = 4
NBUF = 2


def _gather_body(table_hbm, idx_hbm, out_hbm, idx_v, rows_v, sem0, sem1):
    n_rows = idx_hbm.shape[0]
    rows_per_w = n_rows // NW
    n_chunks = rows_per_w // R
    wid = lax.axis_index("s") * NC + lax.axis_index("c")
    base = wid * rows_per_w
    sems = (sem0, sem1)

    def stage(g, b):
        row0 = base + g * R
        pltpu.sync_copy(idx_hbm.at[pl.ds(row0, R)], idx_v.at[b])
        for j in range(R):
            pltpu.async_copy(table_hbm.at[idx_v.at[b, j]], rows_v.at[b, j],
                             sems[b])

    stage(0, 0)

    def pair(t, _):
        for b in range(NBUF):
            g = NBUF * t + b
            nb = 1 - b

            @pl.when(g + 1 < n_chunks)
            def _():
                stage(g + 1, nb)

            pltpu.make_async_copy(table_hbm.at[idx_v.at[b]], rows_v.at[b],
                                  sems[b]).wait()
            pltpu.sync_copy(rows_v.at[b],
                            out_hbm.at[pl.ds(base + g * R, R), :, pl.ds(0, D)])
        return ()

    lax.fori_loop(0, n_chunks // NBUF, pair, (), unroll=False)


def kernel(token_ids, table):
    B, S = token_ids.shape
    V = table.shape[0]
    n_idx = B * S
    assert n_idx % (IW * NW * R * NBUF) == 0
    n_rows = n_idx // IW
    idx2d = token_ids.reshape(n_rows, IW).astype(jnp.int32)
    table_lin = lax.optimization_barrier(table.reshape(-1)).reshape(V, D)

    mesh = plsc.VectorSubcoreMesh(core_axis_name="c", subcore_axis_name="s")
    gather = pl.kernel(
        _gather_body,
        out_type=jax.ShapeDtypeStruct((n_rows, IW, DP), jnp.float32),
        mesh=mesh,
        scratch_types=[
            pltpu.VMEM((NBUF, R, IW), jnp.int32),
            pltpu.VMEM((NBUF, R, IW, D), jnp.float32),
            pltpu.SemaphoreType.DMA,
            pltpu.SemaphoreType.DMA,
        ],
        compiler_params=pltpu.CompilerParams(use_tc_tiling_on_sc=False),
    )
    out = gather(table_lin, idx2d)
    return out[:, :, :D].reshape(B, S, D)
